# baseline (speedup 1.0000x reference)
import jax
import jax.numpy as jnp
from jax import lax
from jax.experimental import pallas as pl
from jax.experimental.pallas import tpu as pltpu

_N_VOCAB = 1000000
_N_EMBED = 64
_BATCH = 16384
_TC_BLK = 512


def _tc_gather_kernel(idx_smem, tbl_any, out_vmem, sem):
    g = pl.program_id(0)

    def fire(j, _):
        i = idx_smem[g * _TC_BLK + j]
        pltpu.make_async_copy(
            tbl_any.at[pl.ds(i, 1)], out_vmem.at[pl.ds(j, 1)], sem
        ).start()
        return ()

    lax.fori_loop(0, _TC_BLK, fire, (), unroll=8)

    def drain(j, _):
        pltpu.make_async_copy(
            tbl_any.at[pl.ds(0, 1)], out_vmem.at[pl.ds(0, 1)], sem
        ).wait()
        return ()

    lax.fori_loop(0, _TC_BLK, drain, (), unroll=8)


@jax.jit
def kernel(input_words, in_embed_weight):
    grid_spec = pltpu.PrefetchScalarGridSpec(
        num_scalar_prefetch=1,
        grid=(_BATCH // _TC_BLK,),
        in_specs=[pl.BlockSpec(memory_space=pltpu.MemorySpace.HBM)],
        out_specs=pl.BlockSpec((_TC_BLK, _N_EMBED), lambda g, idx: (g, 0)),
        scratch_shapes=[pltpu.SemaphoreType.DMA],
    )
    out = pl.pallas_call(
        _tc_gather_kernel,
        grid_spec=grid_spec,
        out_shape=jax.ShapeDtypeStruct((_BATCH, _N_EMBED), jnp.float32),
    )(input_words, in_embed_weight)
    return out


# speedup vs baseline: 1.0334x; 1.0334x over previous
import jax
import jax.numpy as jnp
from jax import lax
from jax.experimental import pallas as pl
from jax.experimental.pallas import tpu as pltpu
from jax.experimental.pallas import tpu_sc as plsc

_N_VOCAB = 1000000
_N_EMBED = 64
_BATCH = 16384

_info = plsc.get_sparse_core_info()
_NC = _info.num_cores
_NS = _info.num_subcores
_NW = _NC * _NS

_S = 7168
_B_PER_W = _S // _NW
_K = 16
_NBATCH = _B_PER_W // _K

_TC_BLK = 512
_T = _BATCH - _S


def _sc_gather_kernel(tbl_hbm, idx_hbm, out_hbm, idx_v, rows_v, sem):
    wid = lax.axis_index("s") * _NC + lax.axis_index("c")
    base = wid * _B_PER_W
    pltpu.sync_copy(idx_hbm.at[wid], idx_v)

    def batch_body(b, _):
        vblk = idx_v[pl.ds(b * _K, _K)]
        for l in range(_K):
            i = vblk[l]
            pltpu.async_copy(
                tbl_hbm.at[pl.ds(i, 1), :],
                rows_v.at[pl.ds(b * _K + l, 1), :],
                sem,
            )
        return ()

    lax.fori_loop(0, _NBATCH, batch_body, (), unroll=False)
    pltpu.make_async_copy(
        tbl_hbm.at[pl.ds(0, _B_PER_W), :], rows_v, sem
    ).wait()
    pltpu.sync_copy(rows_v, out_hbm.at[pl.ds(base, _B_PER_W)])


def _tc_gather_kernel(idx_smem, tbl_hbm, out_vmem, sem):
    g = pl.program_id(0)

    def fire(j, _):
        i = idx_smem[g * _TC_BLK + j]
        pltpu.make_async_copy(
            tbl_hbm.at[pl.ds(i, 1)], out_vmem.at[pl.ds(j, 1)], sem
        ).start()
        return ()

    lax.fori_loop(0, _TC_BLK, fire, (), unroll=16)
    pltpu.make_async_copy(
        tbl_hbm.at[pl.ds(0, _TC_BLK)], out_vmem, sem
    ).wait()


@jax.jit
def kernel(input_words, in_embed_weight):
    idx_sc = input_words[:_S].reshape(_NW, _B_PER_W)
    idx_tc = input_words[_S:]

    mesh = plsc.VectorSubcoreMesh(core_axis_name="c", subcore_axis_name="s")
    out_sc = pl.kernel(
        _sc_gather_kernel,
        mesh=mesh,
        out_type=jax.ShapeDtypeStruct((_S, _N_EMBED), jnp.float32),
        scratch_types=[
            pltpu.VMEM((_B_PER_W,), jnp.int32),
            pltpu.VMEM((_B_PER_W, _N_EMBED), jnp.float32),
            pltpu.SemaphoreType.DMA,
        ],
    )(in_embed_weight, idx_sc)

    grid_spec = pltpu.PrefetchScalarGridSpec(
        num_scalar_prefetch=1,
        grid=(_T // _TC_BLK,),
        in_specs=[pl.BlockSpec(memory_space=pltpu.MemorySpace.HBM)],
        out_specs=pl.BlockSpec((_TC_BLK, _N_EMBED), lambda g, idx: (g, 0)),
        scratch_shapes=[pltpu.SemaphoreType.DMA],
    )
    out_tc = pl.pallas_call(
        _tc_gather_kernel,
        grid_spec=grid_spec,
        out_shape=jax.ShapeDtypeStruct((_T, _N_EMBED), jnp.float32),
    )(idx_tc, in_embed_weight)

    return jnp.concatenate([out_sc, out_tc], axis=0)
